# Initial kernel scaffold; baseline (speedup 1.0000x reference)
import functools

import jax
import jax.numpy as jnp
from jax.experimental import pallas as pl
from jax.experimental.pallas import tpu as pltpu

NUM_EXPERTS = 8
D_MODEL = 1024
HIDDEN = 3 * D_MODEL
LANES = 128

ROUTER_CHUNK = 512
FFN_CHUNK = 512


def _router_body(x_ref, wr_ref, rw_ref, aux_ref, cnt_ref, psum_ref):
    i = pl.program_id(0)
    nsteps = pl.num_programs(0)
    ntok = x_ref.shape[0]

    logits = jnp.dot(x_ref[...], wr_ref[...], preferred_element_type=jnp.float32)
    col = jax.lax.broadcasted_iota(jnp.int32, logits.shape, 1)
    valid = col < NUM_EXPERTS
    logits = jnp.where(valid, logits, -jnp.inf)
    mx = jnp.max(logits, axis=1, keepdims=True)
    ex = jnp.where(valid, jnp.exp(logits - mx), 0.0)
    sm = jnp.sum(ex, axis=1, keepdims=True)
    probs = ex / sm

    v1 = jnp.max(probs, axis=1, keepdims=True)
    i1 = jnp.min(jnp.where((probs == v1) & valid, col, LANES), axis=1, keepdims=True)
    m1 = col == i1
    probs2 = jnp.where(m1, -1.0, probs)
    v2 = jnp.max(probs2, axis=1, keepdims=True)
    i2 = jnp.min(jnp.where((probs2 == v2) & valid, col, LANES), axis=1, keepdims=True)

    denom = v1 + v2
    w1 = v1 / denom
    w2 = v2 / denom
    rw = jnp.where(col == i1, w1, 0.0) + jnp.where(col == i2, w2, 0.0)
    rw_ref[...] = rw

    sel = (col == i1) | (col == i2)
    cnt_part = jnp.sum(sel.astype(jnp.float32), axis=0, keepdims=True)
    psum_part = jnp.sum(probs, axis=0, keepdims=True)

    @pl.when(i == 0)
    def _():
        cnt_ref[...] = cnt_part
        psum_ref[...] = psum_part

    @pl.when(i > 0)
    def _():
        cnt_ref[...] += cnt_part
        psum_ref[...] += psum_part

    @pl.when(i == nsteps - 1)
    def _():
        total = ntok * nsteps
        fi = cnt_ref[...] / total
        pi = psum_ref[...] / total
        aux = 0.01 * NUM_EXPERTS * jnp.sum(fi * pi)
        aux_ref[...] = jnp.full((1, LANES), aux, dtype=jnp.float32)


def _router(x_flat, wr_pad):
    n = x_flat.shape[0]
    nsteps = n // ROUTER_CHUNK
    return pl.pallas_call(
        _router_body,
        grid=(nsteps,),
        in_specs=[
            pl.BlockSpec((ROUTER_CHUNK, D_MODEL), lambda i: (i, 0)),
            pl.BlockSpec((D_MODEL, LANES), lambda i: (0, 0)),
        ],
        out_specs=[
            pl.BlockSpec((ROUTER_CHUNK, LANES), lambda i: (i, 0)),
            pl.BlockSpec((1, LANES), lambda i: (0, 0)),
        ],
        out_shape=[
            jax.ShapeDtypeStruct((n, LANES), jnp.float32),
            jax.ShapeDtypeStruct((1, LANES), jnp.float32),
        ],
        scratch_shapes=[
            pltpu.VMEM((1, LANES), jnp.float32),
            pltpu.VMEM((1, LANES), jnp.float32),
        ],
    )(x_flat, wr_pad)


def _ffn_body(x_ref, rw_ref, w1_ref, w2_ref, out_ref):
    e = pl.program_id(0)
    m = pl.program_id(1)
    rows = pl.ds(m * FFN_CHUNK, FFN_CHUNK)

    xs = x_ref[rows, :]
    h = jnp.dot(xs, w1_ref[0], preferred_element_type=jnp.float32)
    h = h * jax.nn.sigmoid(h)
    o = jnp.dot(h, w2_ref[0], preferred_element_type=jnp.float32)

    col = jax.lax.broadcasted_iota(jnp.int32, (FFN_CHUNK, LANES), 1)
    w = jnp.sum(jnp.where(col == e, rw_ref[rows, :], 0.0), axis=1, keepdims=True)
    contrib = o * w

    @pl.when(e == 0)
    def _():
        out_ref[rows, :] = contrib

    @pl.when(e > 0)
    def _():
        out_ref[rows, :] += contrib


def _ffn(x_flat, rw, w1, w2):
    n = x_flat.shape[0]
    mt = n // FFN_CHUNK
    return pl.pallas_call(
        _ffn_body,
        grid=(NUM_EXPERTS, mt),
        in_specs=[
            pl.BlockSpec((n, D_MODEL), lambda e, m: (0, 0)),
            pl.BlockSpec((n, LANES), lambda e, m: (0, 0)),
            pl.BlockSpec((1, D_MODEL, HIDDEN), lambda e, m: (e, 0, 0)),
            pl.BlockSpec((1, HIDDEN, D_MODEL), lambda e, m: (e, 0, 0)),
        ],
        out_specs=pl.BlockSpec((n, D_MODEL), lambda e, m: (0, 0)),
        out_shape=jax.ShapeDtypeStruct((n, D_MODEL), jnp.float32),
    )(x_flat, rw, w1, w2)


def kernel(x, Wr, W1, W2):
    b, t, c = x.shape
    x_flat = x.reshape(-1, c)
    wr_pad = jnp.pad(Wr, ((0, 0), (0, LANES - NUM_EXPERTS)))
    rw, aux_vec = _router(x_flat, wr_pad)
    out_flat = _ffn(x_flat, rw, W1, W2)
    return out_flat.reshape(b, t, c), aux_vec[0, 0]


# speedup vs baseline: 1.1046x; 1.1046x over previous
import functools

import jax
import jax.numpy as jnp
from jax.experimental import pallas as pl
from jax.experimental.pallas import tpu as pltpu

NUM_EXPERTS = 8
D_MODEL = 1024
HIDDEN = 3 * D_MODEL
LANES = 128

ROUTER_CHUNK = 512
FFN_CHUNK = 512


def _router_body(x_ref, wr_ref, rw_ref, aux_ref, cnt_ref, psum_ref):
    i = pl.program_id(0)
    nsteps = pl.num_programs(0)
    ntok = x_ref.shape[0]

    logits = jnp.dot(x_ref[...], wr_ref[...], preferred_element_type=jnp.float32)
    col = jax.lax.broadcasted_iota(jnp.int32, logits.shape, 1)
    valid = col < NUM_EXPERTS
    logits = jnp.where(valid, logits, -jnp.inf)
    mx = jnp.max(logits, axis=1, keepdims=True)
    ex = jnp.where(valid, jnp.exp(logits - mx), 0.0)
    sm = jnp.sum(ex, axis=1, keepdims=True)
    probs = ex / sm

    v1 = jnp.max(probs, axis=1, keepdims=True)
    i1 = jnp.min(jnp.where((probs == v1) & valid, col, LANES), axis=1, keepdims=True)
    m1 = col == i1
    probs2 = jnp.where(m1, -1.0, probs)
    v2 = jnp.max(probs2, axis=1, keepdims=True)
    i2 = jnp.min(jnp.where((probs2 == v2) & valid, col, LANES), axis=1, keepdims=True)

    denom = v1 + v2
    w1 = v1 / denom
    w2 = v2 / denom
    rw = jnp.where(col == i1, w1, 0.0) + jnp.where(col == i2, w2, 0.0)
    rw_ref[...] = rw

    sel = (col == i1) | (col == i2)
    cnt_part = jnp.sum(sel.astype(jnp.float32), axis=0, keepdims=True)
    psum_part = jnp.sum(probs, axis=0, keepdims=True)

    @pl.when(i == 0)
    def _():
        cnt_ref[...] = cnt_part
        psum_ref[...] = psum_part

    @pl.when(i > 0)
    def _():
        cnt_ref[...] += cnt_part
        psum_ref[...] += psum_part

    @pl.when(i == nsteps - 1)
    def _():
        total = ntok * nsteps
        fi = cnt_ref[...] / total
        pi = psum_ref[...] / total
        aux = 0.01 * NUM_EXPERTS * jnp.sum(fi * pi)
        aux_ref[...] = jnp.full((1, LANES), aux, dtype=jnp.float32)


def _router(x_flat, wr_pad):
    n = x_flat.shape[0]
    nsteps = n // ROUTER_CHUNK
    return pl.pallas_call(
        _router_body,
        grid=(nsteps,),
        in_specs=[
            pl.BlockSpec((ROUTER_CHUNK, D_MODEL), lambda i: (i, 0)),
            pl.BlockSpec((D_MODEL, LANES), lambda i: (0, 0)),
        ],
        out_specs=[
            pl.BlockSpec((ROUTER_CHUNK, LANES), lambda i: (i, 0)),
            pl.BlockSpec((1, LANES), lambda i: (0, 0)),
        ],
        out_shape=[
            jax.ShapeDtypeStruct((n, LANES), jnp.float32),
            jax.ShapeDtypeStruct((1, LANES), jnp.float32),
        ],
        scratch_shapes=[
            pltpu.VMEM((1, LANES), jnp.float32),
            pltpu.VMEM((1, LANES), jnp.float32),
        ],
    )(x_flat, wr_pad)


HBLOCK = 1536
HT = HIDDEN // HBLOCK


def _ffn_body(x_ref, rw_ref, w1_ref, w2_ref, out_ref, acc_ref):
    e = pl.program_id(1)
    h = pl.program_id(2)

    hh = jnp.dot(x_ref[...], w1_ref[0], preferred_element_type=jnp.float32)
    hh = hh * jax.nn.sigmoid(hh)
    part = jnp.dot(hh, w2_ref[0], preferred_element_type=jnp.float32)

    col = jax.lax.broadcasted_iota(jnp.int32, (FFN_CHUNK, LANES), 1)
    w = jnp.sum(jnp.where(col == e, rw_ref[...], 0.0), axis=1, keepdims=True)
    contrib = part * w

    @pl.when((e == 0) & (h == 0))
    def _():
        acc_ref[...] = contrib

    @pl.when((e > 0) | (h > 0))
    def _():
        acc_ref[...] += contrib

    @pl.when((e == NUM_EXPERTS - 1) & (h == HT - 1))
    def _():
        out_ref[...] = acc_ref[...]


def _ffn(x_flat, rw, w1, w2):
    n = x_flat.shape[0]
    mt = n // FFN_CHUNK
    return pl.pallas_call(
        _ffn_body,
        grid=(mt, NUM_EXPERTS, HT),
        in_specs=[
            pl.BlockSpec((FFN_CHUNK, D_MODEL), lambda m, e, h: (m, 0)),
            pl.BlockSpec((FFN_CHUNK, LANES), lambda m, e, h: (m, 0)),
            pl.BlockSpec((1, D_MODEL, HBLOCK), lambda m, e, h: (e, 0, h)),
            pl.BlockSpec((1, HBLOCK, D_MODEL), lambda m, e, h: (e, h, 0)),
        ],
        out_specs=pl.BlockSpec((FFN_CHUNK, D_MODEL), lambda m, e, h: (m, 0)),
        out_shape=jax.ShapeDtypeStruct((n, D_MODEL), jnp.float32),
        scratch_shapes=[pltpu.VMEM((FFN_CHUNK, D_MODEL), jnp.float32)],
    )(x_flat, rw, w1, w2)


def kernel(x, Wr, W1, W2):
    b, t, c = x.shape
    x_flat = x.reshape(-1, c)
    wr_pad = jnp.pad(Wr, ((0, 0), (0, LANES - NUM_EXPERTS)))
    rw, aux_vec = _router(x_flat, wr_pad)
    out_flat = _ffn(x_flat, rw, W1, W2)
    return out_flat.reshape(b, t, c), aux_vec[0, 0]


# speedup vs baseline: 1.4974x; 1.3556x over previous
import functools

import jax
import jax.numpy as jnp
from jax import lax
from jax.experimental import pallas as pl
from jax.experimental.pallas import tpu as pltpu
from jax.experimental.pallas import tpu_sc as plsc

NUM_EXPERTS = 8
D_MODEL = 1024
HIDDEN = 3 * D_MODEL
LANES = 128

ROUTER_CHUNK = 512
M = 512
NC, NS, L = 2, 16, 16
NW = NC * NS


def _router_body(x_ref, wr_ref, mi_ref, mf_ref, cnt_ref, aux_ref,
                 carry_ref, psum_ref):
    i = pl.program_id(0)
    nsteps = pl.num_programs(0)
    ntok = x_ref.shape[0]

    logits = jnp.dot(x_ref[...], wr_ref[...], preferred_element_type=jnp.float32)
    col = jax.lax.broadcasted_iota(jnp.int32, logits.shape, 1)
    valid = col < NUM_EXPERTS
    logits = jnp.where(valid, logits, -jnp.inf)
    mx = jnp.max(logits, axis=1, keepdims=True)
    ex = jnp.where(valid, jnp.exp(logits - mx), 0.0)
    sm = jnp.sum(ex, axis=1, keepdims=True)
    probs = ex / sm

    v1 = jnp.max(probs, axis=1, keepdims=True)
    i1 = jnp.min(jnp.where((probs == v1) & valid, col, LANES), axis=1, keepdims=True)
    m1 = col == i1
    probs2 = jnp.where(m1, -1.0, probs)
    v2 = jnp.max(probs2, axis=1, keepdims=True)
    i2 = jnp.min(jnp.where((probs2 == v2) & valid, col, LANES), axis=1, keepdims=True)
    m2 = col == i2

    denom = v1 + v2
    w0 = v1 / denom
    w1 = v2 / denom

    @pl.when(i == 0)
    def _():
        carry_ref[...] = jnp.zeros((1, LANES), jnp.float32)
        psum_ref[...] = jnp.zeros((1, LANES), jnp.float32)

    oh0 = m1.astype(jnp.float32)
    oh1 = m2.astype(jnp.float32)
    ohsum = oh0 + oh1
    rr = jax.lax.broadcasted_iota(jnp.int32, (ntok, ntok), 0)
    cc = jax.lax.broadcasted_iota(jnp.int32, (ntok, ntok), 1)
    ltri = (rr > cc).astype(jnp.float32)
    s_cum = jnp.dot(ltri, ohsum, preferred_element_type=jnp.float32)

    carry = carry_ref[...]
    rank0 = jnp.sum((s_cum + carry) * oh0, axis=1, keepdims=True).astype(jnp.int32)
    rank1 = jnp.sum((s_cum + carry) * oh1, axis=1, keepdims=True).astype(jnp.int32)
    carry_ref[...] = carry + jnp.sum(ohsum, axis=0, keepdims=True)
    psum_ref[...] += jnp.sum(probs, axis=0, keepdims=True)

    mi = (jnp.where(col == 0, i1, 0) + jnp.where(col == 1, i2, 0)
          + jnp.where(col == 2, rank0, 0) + jnp.where(col == 3, rank1, 0))
    mi_ref[...] = mi
    mf_ref[...] = jnp.where(col == 0, w0, 0.0) + jnp.where(col == 1, w1, 0.0)

    @pl.when(i == nsteps - 1)
    def _():
        total = ntok * nsteps
        cnt = carry_ref[...]
        cnt_ref[...] = cnt
        fi = cnt / total
        pi = psum_ref[...] / total
        aux = 0.01 * NUM_EXPERTS * jnp.sum(fi * pi)
        aux_ref[...] = jnp.full((1, LANES), aux, dtype=jnp.float32)


def _router(x_flat, wr_pad):
    n = x_flat.shape[0]
    nsteps = n // ROUTER_CHUNK
    return pl.pallas_call(
        _router_body,
        grid=(nsteps,),
        in_specs=[
            pl.BlockSpec((ROUTER_CHUNK, D_MODEL), lambda i: (i, 0)),
            pl.BlockSpec((D_MODEL, LANES), lambda i: (0, 0)),
        ],
        out_specs=[
            pl.BlockSpec((ROUTER_CHUNK, LANES), lambda i: (i, 0)),
            pl.BlockSpec((ROUTER_CHUNK, LANES), lambda i: (i, 0)),
            pl.BlockSpec((1, LANES), lambda i: (0, 0)),
            pl.BlockSpec((1, LANES), lambda i: (0, 0)),
        ],
        out_shape=[
            jax.ShapeDtypeStruct((n, LANES), jnp.int32),
            jax.ShapeDtypeStruct((n, LANES), jnp.float32),
            jax.ShapeDtypeStruct((1, LANES), jnp.float32),
            jax.ShapeDtypeStruct((1, LANES), jnp.float32),
        ],
        scratch_shapes=[
            pltpu.VMEM((1, LANES), jnp.float32),
            pltpu.VMEM((1, LANES), jnp.float32),
        ],
    )(x_flat, wr_pad)


def _dispatch(x_flat, e0, e1, r0, r1, off16):
    n = x_flat.shape[0]
    tpw = n // NW
    sub = 64

    @functools.partial(
        pl.kernel,
        out_type=[
            jax.ShapeDtypeStruct((2 * n, D_MODEL), jnp.float32),
            jax.ShapeDtypeStruct((n,), jnp.int32),
            jax.ShapeDtypeStruct((n,), jnp.int32),
        ],
        mesh=plsc.VectorSubcoreMesh(core_axis_name="c", subcore_axis_name="s"),
        scratch_types=[
            pltpu.VMEM((16,), jnp.int32),
            pltpu.VMEM((sub,), jnp.int32),
            pltpu.VMEM((sub,), jnp.int32),
            pltpu.VMEM((sub,), jnp.int32),
            pltpu.VMEM((sub,), jnp.int32),
            pltpu.VMEM((sub,), jnp.int32),
            pltpu.VMEM((sub,), jnp.int32),
            pltpu.VMEM((sub, D_MODEL), jnp.float32),
            pltpu.SemaphoreType.DMA,
            pltpu.SemaphoreType.DMA,
        ],
    )
    def k(x_hbm, e0_hbm, e1_hbm, r0_hbm, r1_hbm, offs_hbm,
          xs_hbm, p0_hbm, p1_hbm,
          off_v, e0_v, e1_v, r0_v, r1_v, idx0_v, idx1_v, xv, sem0, sem1):
        wid = lax.axis_index("s") * NC + lax.axis_index("c")
        base = wid * tpw
        pltpu.sync_copy(offs_hbm, off_v)
        for j in range(tpw // sub):
            tb = base + j * sub
            pltpu.sync_copy(e0_hbm.at[pl.ds(tb, sub)], e0_v)
            pltpu.sync_copy(e1_hbm.at[pl.ds(tb, sub)], e1_v)
            pltpu.sync_copy(r0_hbm.at[pl.ds(tb, sub)], r0_v)
            pltpu.sync_copy(r1_hbm.at[pl.ds(tb, sub)], r1_v)
            off_reg = off_v[...]
            for s in range(sub // L):
                sl = pl.ds(s * L, L)
                ev0 = e0_v[sl]
                ev1 = e1_v[sl]
                acc0 = jnp.zeros((L,), jnp.int32)
                acc1 = jnp.zeros((L,), jnp.int32)
                for kk in range(NUM_EXPERTS):
                    off_k = off_reg[kk]
                    acc0 = acc0 + jnp.where(ev0 == kk, off_k, 0)
                    acc1 = acc1 + jnp.where(ev1 == kk, off_k, 0)
                idx0_v[sl] = acc0 + r0_v[sl]
                idx1_v[sl] = acc1 + r1_v[sl]
            pltpu.sync_copy(x_hbm.at[pl.ds(tb, sub)], xv)
            cp0 = pltpu.async_copy(xv, xs_hbm.at[idx0_v], sem0)
            cp1 = pltpu.async_copy(xv, xs_hbm.at[idx1_v], sem1)
            cp0.wait()
            cp1.wait()
            pltpu.sync_copy(idx0_v, p0_hbm.at[pl.ds(tb, sub)])
            pltpu.sync_copy(idx1_v, p1_hbm.at[pl.ds(tb, sub)])

    return k(x_flat, e0, e1, r0, r1, off16)


def _gffn_body(se_ref, sm_ref, sa_ref, off_ref, xs_ref, w1_ref, w2_ref, out_ref):
    g = pl.program_id(0)
    e = se_ref[g]
    m = sm_ref[g]
    prev_m = sm_ref[jnp.maximum(g - 1, 0)]
    first = (g == 0) | (m != prev_m)

    @pl.when(sa_ref[g] == 1)
    def _():
        xb = xs_ref[...].astype(jnp.bfloat16)
        h = jnp.dot(xb, w1_ref[0], preferred_element_type=jnp.float32)
        h = h * jax.nn.sigmoid(h)
        o = jnp.dot(h.astype(jnp.bfloat16), w2_ref[0],
                    preferred_element_type=jnp.float32)
        rowg = m * M + jax.lax.broadcasted_iota(jnp.int32, (M, 1), 0)
        mask = (rowg >= off_ref[e]) & (rowg < off_ref[e + 1])
        contrib = jnp.where(mask, o, 0.0)

        @pl.when(first)
        def _():
            out_ref[...] = contrib

        @pl.when(jnp.logical_not(first))
        def _():
            out_ref[...] += contrib


def _gffn(xs, w1b, w2b, step_e, step_m, step_act, off9, nt):
    na = 2 * NUM_EXPERTS * ROUTER_CHUNK
    rows = xs.shape[0]
    grid_spec = pltpu.PrefetchScalarGridSpec(
        num_scalar_prefetch=4,
        grid=(nt,),
        in_specs=[
            pl.BlockSpec((M, D_MODEL), lambda g, se, sm, sa, off: (sm[g], 0)),
            pl.BlockSpec((1, D_MODEL, HIDDEN), lambda g, se, sm, sa, off: (se[g], 0, 0)),
            pl.BlockSpec((1, HIDDEN, D_MODEL), lambda g, se, sm, sa, off: (se[g], 0, 0)),
        ],
        out_specs=pl.BlockSpec((M, D_MODEL), lambda g, se, sm, sa, off: (sm[g], 0)),
    )
    return pl.pallas_call(
        _gffn_body,
        grid_spec=grid_spec,
        out_shape=jax.ShapeDtypeStruct((rows, D_MODEL), jnp.float32),
    )(step_e, step_m, step_act, off9, xs, w1b, w2b)


def _combine(ys, p0, p1, w0, w1):
    n = p0.shape[0]
    tpw = n // NW
    sub = 16

    @functools.partial(
        pl.kernel,
        out_type=jax.ShapeDtypeStruct((n, D_MODEL), jnp.float32),
        mesh=plsc.VectorSubcoreMesh(core_axis_name="c", subcore_axis_name="s"),
        scratch_types=[
            pltpu.VMEM((sub,), jnp.int32),
            pltpu.VMEM((sub,), jnp.int32),
            pltpu.VMEM((sub,), jnp.float32),
            pltpu.VMEM((sub,), jnp.float32),
            pltpu.VMEM((sub, D_MODEL), jnp.float32),
            pltpu.VMEM((sub, D_MODEL), jnp.float32),
            pltpu.VMEM((sub, D_MODEL), jnp.float32),
            pltpu.SemaphoreType.DMA,
            pltpu.SemaphoreType.DMA,
        ],
    )
    def k(ys_hbm, p0_hbm, p1_hbm, w0_hbm, w1_hbm, out_hbm,
          p0_v, p1_v, w0_v, w1_v, ya, yb, ov, sem0, sem1):
        wid = lax.axis_index("s") * NC + lax.axis_index("c")
        base = wid * tpw

        def jbody(j, carry):
            tb = base + j * sub
            pltpu.sync_copy(p0_hbm.at[pl.ds(tb, sub)], p0_v)
            pltpu.sync_copy(p1_hbm.at[pl.ds(tb, sub)], p1_v)
            pltpu.sync_copy(w0_hbm.at[pl.ds(tb, sub)], w0_v)
            pltpu.sync_copy(w1_hbm.at[pl.ds(tb, sub)], w1_v)
            cpa = pltpu.async_copy(ys_hbm.at[p0_v], ya, sem0)
            cpb = pltpu.async_copy(ys_hbm.at[p1_v], yb, sem1)
            cpa.wait()
            cpb.wait()
            wreg_a = w0_v[...]
            wreg_b = w1_v[...]
            for t in range(sub):
                wa = wreg_a[t]
                wb = wreg_b[t]
                for jj in range(D_MODEL // L):
                    sl = pl.ds(jj * L, L)
                    ov[t, sl] = ya[t, sl] * wa + yb[t, sl] * wb
            pltpu.sync_copy(ov, out_hbm.at[pl.ds(tb, sub)])
            return carry

        lax.fori_loop(0, tpw // sub, jbody, 0)

    return k(ys, p0, p1, w0, w1)


def kernel(x, Wr, W1, W2):
    b, t, c = x.shape
    n = b * t
    x_flat = x.reshape(n, c)
    wr_pad = jnp.pad(Wr, ((0, 0), (0, LANES - NUM_EXPERTS)))

    mi, mf, cnt_row, aux_vec = _router(x_flat, wr_pad)
    e0 = mi[:, 0]
    e1 = mi[:, 1]
    r0 = mi[:, 2]
    r1 = mi[:, 3]
    w0 = mf[:, 0]
    w1 = mf[:, 1]
    counts = cnt_row[0, :NUM_EXPERTS].astype(jnp.int32)

    off9 = jnp.concatenate([jnp.zeros((1,), jnp.int32), jnp.cumsum(counts)])
    off16 = jnp.pad(off9, (0, 16 - off9.shape[0]))

    mt = (2 * n) // M
    nt = mt + NUM_EXPERTS - 1
    lo_row = off9[:NUM_EXPERTS]
    hi_row = off9[1:]
    tile_lo = lo_row // M
    tile_last = jnp.where(counts > 0, (hi_row - 1) // M, tile_lo)
    nact = jnp.where(counts > 0, tile_last - tile_lo + 1, 0)
    gstart = jnp.concatenate([jnp.zeros((1,), jnp.int32),
                              jnp.cumsum(nact)[:-1]])
    g = jnp.arange(nt, dtype=jnp.int32)
    e_of_g = jnp.sum((g[:, None] >= gstart[None, :]).astype(jnp.int32), axis=1) - 1
    within = g - gstart[e_of_g]
    act = within < nact[e_of_g]
    m_of_g = jnp.where(act, tile_lo[e_of_g] + within, mt - 1)
    step_e = e_of_g.astype(jnp.int32)
    step_m = m_of_g.astype(jnp.int32)
    step_act = act.astype(jnp.int32)

    xs, p0, p1 = _dispatch(x_flat, e0, e1, r0, r1, off16)
    ys = _gffn(xs, W1.astype(jnp.bfloat16), W2.astype(jnp.bfloat16),
               step_e, step_m, step_act, off9, nt)
    out_flat = _combine(ys, p0, p1, w0, w1)
    return out_flat.reshape(b, t, c), aux_vec[0, 0]
